# baseline (speedup 1.0000x reference)
import jax
import jax.numpy as jnp
from jax.experimental import pallas as pl
from jax.experimental.pallas import tpu as pltpu

_N, _D, _H, _C = 4096, 512, 512, 64
_BM = 512


def _stage1(feat_ref, w0_ref, b0_ref, w1_ref, adj_ref, adjb_ref, y1_ref, y0_scr):
    i = pl.program_id(0)

    @pl.when(i == 0)
    def _():
        y0_scr[...] = jnp.dot(
            feat_ref[...], w0_ref[...], preferred_element_type=jnp.float32
        ).astype(jnp.bfloat16)

    a = adj_ref[...].astype(jnp.bfloat16)
    adjb_ref[...] = a
    h = jnp.dot(a, y0_scr[...], preferred_element_type=jnp.float32)
    h = jnp.maximum(h + b0_ref[...], 0.0).astype(jnp.bfloat16)
    y1_ref[...] = jnp.dot(
        h, w1_ref[...], preferred_element_type=jnp.float32
    ).astype(jnp.bfloat16)


def _stage2(adjb_ref, y1_ref, b1_ref, wc_ref, y2_ref):
    h = jnp.dot(adjb_ref[...], y1_ref[...], preferred_element_type=jnp.float32)
    h = jnp.maximum(h + b1_ref[...], 0.0).astype(jnp.bfloat16)
    y2_ref[...] = jnp.dot(
        h, wc_ref[...], preferred_element_type=jnp.float32
    ).astype(jnp.bfloat16)


def _stage3(adjb_ref, y2_ref, bc_ref, out_ref):
    out_ref[...] = (
        jnp.dot(adjb_ref[...], y2_ref[...], preferred_element_type=jnp.float32)
        + bc_ref[...]
    )


def kernel(features, adj, W0, b0, W1, b1, Wc, bc):
    nblk = _N // _BM
    feat_b = features.astype(jnp.bfloat16)
    w0_b = W0.astype(jnp.bfloat16)
    w1_b = W1.astype(jnp.bfloat16)
    wc_b = Wc.astype(jnp.bfloat16)
    b0_2 = b0.reshape(1, _H)
    b1_2 = b1.reshape(1, _H)
    bc_2 = bc.reshape(1, _C)

    full = lambda shape: pl.BlockSpec(shape, lambda i: (0, 0))
    rows = lambda shape: pl.BlockSpec(shape, lambda i: (i, 0))

    adjb, y1 = pl.pallas_call(
        _stage1,
        grid=(nblk,),
        in_specs=[
            full((_N, _D)),
            full((_D, _H)),
            full((1, _H)),
            full((_H, _H)),
            rows((_BM, _N)),
        ],
        out_specs=[rows((_BM, _N)), rows((_BM, _H))],
        out_shape=[
            jax.ShapeDtypeStruct((_N, _N), jnp.bfloat16),
            jax.ShapeDtypeStruct((_N, _H), jnp.bfloat16),
        ],
        scratch_shapes=[pltpu.VMEM((_N, _H), jnp.bfloat16)],
        compiler_params=pltpu.CompilerParams(
            dimension_semantics=("arbitrary",),
        ),
    )(feat_b, w0_b, b0_2, w1_b, adj)

    y2 = pl.pallas_call(
        _stage2,
        grid=(nblk,),
        in_specs=[rows((_BM, _N)), full((_N, _H)), full((1, _H)), full((_H, _C))],
        out_specs=rows((_BM, _C)),
        out_shape=jax.ShapeDtypeStruct((_N, _C), jnp.bfloat16),
        compiler_params=pltpu.CompilerParams(
            dimension_semantics=("arbitrary",),
        ),
    )(adjb, y1, b1_2, wc_b)

    out = pl.pallas_call(
        _stage3,
        grid=(nblk,),
        in_specs=[rows((_BM, _N)), full((_N, _C)), full((1, _C))],
        out_specs=rows((_BM, _C)),
        out_shape=jax.ShapeDtypeStruct((_N, _C), jnp.float32),
        compiler_params=pltpu.CompilerParams(
            dimension_semantics=("arbitrary",),
        ),
    )(adjb, y2, bc_2)

    return out


# speedup vs baseline: 1.0778x; 1.0778x over previous
import jax
import jax.numpy as jnp
from jax.experimental import pallas as pl
from jax.experimental.pallas import tpu as pltpu

_N, _D, _H, _C = 4096, 512, 512, 64
_BM = 256
_NB = _N // _BM


def _y0(feat_ref, w0_ref, y0_ref):
    y0_ref[...] = jnp.dot(
        feat_ref[...], w0_ref[...], preferred_element_type=jnp.float32
    ).astype(jnp.bfloat16)


def _mega(y0_ref, b0_ref, w1_ref, b1_ref, wc_ref, bc_ref, adj_ref,
          out_ref, adjb_scr, y1_scr, y2_scr):
    i = pl.program_id(0)

    @pl.when(i < _NB)
    def _():
        a = adj_ref[...].astype(jnp.bfloat16)
        adjb_scr[pl.ds(i * _BM, _BM), :] = a
        h = jnp.dot(a, y0_ref[...], preferred_element_type=jnp.float32)
        h = jnp.maximum(h + b0_ref[...], 0.0).astype(jnp.bfloat16)
        y1_scr[pl.ds(i * _BM, _BM), :] = jnp.dot(
            h, w1_ref[...], preferred_element_type=jnp.float32
        ).astype(jnp.bfloat16)

    @pl.when((i >= _NB) & (i < 2 * _NB))
    def _():
        j = i - _NB
        a = adjb_scr[pl.ds(j * _BM, _BM), :]
        h = jnp.dot(a, y1_scr[...], preferred_element_type=jnp.float32)
        h = jnp.maximum(h + b1_ref[...], 0.0).astype(jnp.bfloat16)
        y2_scr[pl.ds(j * _BM, _BM), :] = jnp.dot(
            h, wc_ref[...], preferred_element_type=jnp.float32
        ).astype(jnp.bfloat16)

    @pl.when(i >= 2 * _NB)
    def _():
        j = i - 2 * _NB
        a = adjb_scr[pl.ds(j * _BM, _BM), :]
        out_ref[...] = (
            jnp.dot(a, y2_scr[...], preferred_element_type=jnp.float32)
            + bc_ref[...]
        )


def kernel(features, adj, W0, b0, W1, b1, Wc, bc):
    feat_b = features.astype(jnp.bfloat16)
    w0_b = W0.astype(jnp.bfloat16)
    w1_b = W1.astype(jnp.bfloat16)
    wc_b = Wc.astype(jnp.bfloat16)
    b0_2 = b0.reshape(1, _H)
    b1_2 = b1.reshape(1, _H)
    bc_2 = bc.reshape(1, _C)

    full = lambda shape: pl.BlockSpec(shape, lambda i: (0, 0))

    y0 = pl.pallas_call(
        _y0,
        in_specs=[pl.BlockSpec((_N, _D), lambda: (0, 0)),
                  pl.BlockSpec((_D, _H), lambda: (0, 0))],
        out_specs=pl.BlockSpec((_N, _H), lambda: (0, 0)),
        out_shape=jax.ShapeDtypeStruct((_N, _H), jnp.bfloat16),
    )(feat_b, w0_b)

    out = pl.pallas_call(
        _mega,
        grid=(3 * _NB,),
        in_specs=[
            full((_N, _H)),
            full((1, _H)),
            full((_H, _H)),
            full((1, _H)),
            full((_H, _C)),
            full((1, _C)),
            pl.BlockSpec((_BM, _N), lambda i: (jnp.minimum(i, _NB - 1), 0)),
        ],
        out_specs=pl.BlockSpec(
            (_BM, _C), lambda i: (jnp.maximum(i - 2 * _NB, 0), 0)
        ),
        out_shape=jax.ShapeDtypeStruct((_N, _C), jnp.float32),
        scratch_shapes=[
            pltpu.VMEM((_N, _N), jnp.bfloat16),
            pltpu.VMEM((_N, _H), jnp.bfloat16),
            pltpu.VMEM((_N, _C), jnp.bfloat16),
        ],
        compiler_params=pltpu.CompilerParams(
            dimension_semantics=("arbitrary",),
        ),
    )(y0, b0_2, w1_b, b1_2, wc_b, bc_2, adj)

    return out
